# Initial kernel scaffold; baseline (speedup 1.0000x reference)
import jax
import jax.numpy as jnp
from jax.experimental import pallas as pl


def kernel(inputs, W, b):
    raise NotImplementedError("write your pallas kernel here")


# speedup vs baseline: 1.3383x; 1.3383x over previous
import functools

import jax
import jax.numpy as jnp
from jax import lax
from jax.experimental import pallas as pl

NUM_TOKENS = 16384
D_MODEL = 2048
NUM_EXPERTS = 64
TOP_K = 2

BT = 512


def _router_kernel(x_ref, w_ref, b_ref, probs_ref, topk_ref, idx_ref):
    x = x_ref[...]
    w = w_ref[...]
    logits = jnp.dot(x, w, preferred_element_type=jnp.float32) + b_ref[...]

    m = jnp.max(logits, axis=-1, keepdims=True)
    e = jnp.exp(logits - m)
    s = jnp.sum(e, axis=-1, keepdims=True)
    probs_ref[...] = e / s

    iota = lax.broadcasted_iota(jnp.int32, logits.shape, 1)
    neg = jnp.float32(-jnp.inf)

    m1 = jnp.max(logits, axis=-1, keepdims=True)
    i1 = jnp.min(
        jnp.where(logits == m1, iota, NUM_EXPERTS), axis=-1, keepdims=True
    )
    masked = jnp.where(iota == i1, neg, logits)
    m2 = jnp.max(masked, axis=-1, keepdims=True)
    i2 = jnp.min(
        jnp.where(masked == m2, iota, NUM_EXPERTS), axis=-1, keepdims=True
    )

    e2 = jnp.exp(m2 - m1)
    p1 = 1.0 / (1.0 + e2)
    p2 = e2 / (1.0 + e2)

    topk_ref[...] = jnp.concatenate([p1, p2], axis=-1)
    idx_ref[...] = jnp.concatenate([i1, i2], axis=-1)


@jax.jit
def kernel(inputs, W, b):
    b2 = b.reshape(1, NUM_EXPERTS)
    grid = (NUM_TOKENS // BT,)
    probs, topk, idx = pl.pallas_call(
        _router_kernel,
        grid=grid,
        in_specs=[
            pl.BlockSpec((BT, D_MODEL), lambda i: (i, 0)),
            pl.BlockSpec((D_MODEL, NUM_EXPERTS), lambda i: (0, 0)),
            pl.BlockSpec((1, NUM_EXPERTS), lambda i: (0, 0)),
        ],
        out_specs=[
            pl.BlockSpec((BT, NUM_EXPERTS), lambda i: (i, 0)),
            pl.BlockSpec((BT, TOP_K), lambda i: (i, 0)),
            pl.BlockSpec((BT, TOP_K), lambda i: (i, 0)),
        ],
        out_shape=[
            jax.ShapeDtypeStruct((NUM_TOKENS, NUM_EXPERTS), jnp.float32),
            jax.ShapeDtypeStruct((NUM_TOKENS, TOP_K), jnp.float32),
            jax.ShapeDtypeStruct((NUM_TOKENS, TOP_K), jnp.int32),
        ],
    )(inputs, W, b2)
    return (topk, idx, probs)


# speedup vs baseline: 1.5352x; 1.1472x over previous
import functools

import jax
import jax.numpy as jnp
from jax import lax
from jax.experimental import pallas as pl

NUM_TOKENS = 16384
D_MODEL = 2048
NUM_EXPERTS = 64
TOP_K = 2

BT = 1024


def _router_kernel(x_ref, w_ref, b_ref, probs_ref, topk_ref, idx_ref):
    x = x_ref[...]
    w = w_ref[...]
    logits = jnp.dot(x, w, preferred_element_type=jnp.float32) + b_ref[...]

    m = jnp.max(logits, axis=-1, keepdims=True)
    e = jnp.exp(logits - m)
    s = jnp.sum(e, axis=-1, keepdims=True)
    probs_ref[...] = e / s

    iota = lax.broadcasted_iota(jnp.int32, logits.shape, 1)
    neg = jnp.float32(-jnp.inf)

    m1 = jnp.max(logits, axis=-1, keepdims=True)
    i1 = jnp.min(
        jnp.where(logits == m1, iota, NUM_EXPERTS), axis=-1, keepdims=True
    )
    masked = jnp.where(iota == i1, neg, logits)
    m2 = jnp.max(masked, axis=-1, keepdims=True)
    i2 = jnp.min(
        jnp.where(masked == m2, iota, NUM_EXPERTS), axis=-1, keepdims=True
    )

    e2 = jnp.exp(m2 - m1)
    p1 = 1.0 / (1.0 + e2)
    p2 = e2 / (1.0 + e2)

    topk_ref[...] = jnp.concatenate([p1, p2], axis=-1)
    idx_ref[...] = jnp.concatenate([i1, i2], axis=-1)


@jax.jit
def kernel(inputs, W, b):
    b2 = b.reshape(1, NUM_EXPERTS)
    grid = (NUM_TOKENS // BT,)
    probs, topk, idx = pl.pallas_call(
        _router_kernel,
        grid=grid,
        in_specs=[
            pl.BlockSpec((BT, D_MODEL), lambda i: (i, 0)),
            pl.BlockSpec((D_MODEL, NUM_EXPERTS), lambda i: (0, 0)),
            pl.BlockSpec((1, NUM_EXPERTS), lambda i: (0, 0)),
        ],
        out_specs=[
            pl.BlockSpec((BT, NUM_EXPERTS), lambda i: (i, 0)),
            pl.BlockSpec((BT, TOP_K), lambda i: (i, 0)),
            pl.BlockSpec((BT, TOP_K), lambda i: (i, 0)),
        ],
        out_shape=[
            jax.ShapeDtypeStruct((NUM_TOKENS, NUM_EXPERTS), jnp.float32),
            jax.ShapeDtypeStruct((NUM_TOKENS, TOP_K), jnp.float32),
            jax.ShapeDtypeStruct((NUM_TOKENS, TOP_K), jnp.int32),
        ],
    )(inputs, W, b2)
    return (topk, idx, probs)


# speedup vs baseline: 1.5728x; 1.0245x over previous
import functools

import jax
import jax.numpy as jnp
from jax import lax
from jax.experimental import pallas as pl

NUM_TOKENS = 16384
D_MODEL = 2048
NUM_EXPERTS = 64
TOP_K = 2

BT = 2048


def _router_kernel(x_ref, w_ref, b_ref, probs_ref, topk_ref, idx_ref):
    x = x_ref[...]
    w = w_ref[...]
    logits = jnp.dot(x, w, preferred_element_type=jnp.float32) + b_ref[...]

    m = jnp.max(logits, axis=-1, keepdims=True)
    e = jnp.exp(logits - m)
    s = jnp.sum(e, axis=-1, keepdims=True)
    probs_ref[...] = e / s

    iota = lax.broadcasted_iota(jnp.int32, logits.shape, 1)
    neg = jnp.float32(-jnp.inf)

    m1 = jnp.max(logits, axis=-1, keepdims=True)
    i1 = jnp.min(
        jnp.where(logits == m1, iota, NUM_EXPERTS), axis=-1, keepdims=True
    )
    masked = jnp.where(iota == i1, neg, logits)
    m2 = jnp.max(masked, axis=-1, keepdims=True)
    i2 = jnp.min(
        jnp.where(masked == m2, iota, NUM_EXPERTS), axis=-1, keepdims=True
    )

    e2 = jnp.exp(m2 - m1)
    p1 = 1.0 / (1.0 + e2)
    p2 = e2 / (1.0 + e2)

    topk_ref[...] = jnp.concatenate([p1, p2], axis=-1)
    idx_ref[...] = jnp.concatenate([i1, i2], axis=-1)


@jax.jit
def kernel(inputs, W, b):
    b2 = b.reshape(1, NUM_EXPERTS)
    grid = (NUM_TOKENS // BT,)
    probs, topk, idx = pl.pallas_call(
        _router_kernel,
        grid=grid,
        in_specs=[
            pl.BlockSpec((BT, D_MODEL), lambda i: (i, 0)),
            pl.BlockSpec((D_MODEL, NUM_EXPERTS), lambda i: (0, 0)),
            pl.BlockSpec((1, NUM_EXPERTS), lambda i: (0, 0)),
        ],
        out_specs=[
            pl.BlockSpec((BT, NUM_EXPERTS), lambda i: (i, 0)),
            pl.BlockSpec((BT, TOP_K), lambda i: (i, 0)),
            pl.BlockSpec((BT, TOP_K), lambda i: (i, 0)),
        ],
        out_shape=[
            jax.ShapeDtypeStruct((NUM_TOKENS, NUM_EXPERTS), jnp.float32),
            jax.ShapeDtypeStruct((NUM_TOKENS, TOP_K), jnp.float32),
            jax.ShapeDtypeStruct((NUM_TOKENS, TOP_K), jnp.int32),
        ],
    )(inputs, W, b2)
    return (topk, idx, probs)


# speedup vs baseline: 1.5735x; 1.0004x over previous
import functools

import jax
import jax.numpy as jnp
from jax import lax
from jax.experimental import pallas as pl

NUM_TOKENS = 16384
D_MODEL = 2048
NUM_EXPERTS = 64
TOP_K = 2

BT = 2048
NSPLIT = 2
BTS = BT // NSPLIT


def _top2(logits, probs_ref_slice, topk_ref_slice, idx_ref_slice):
    m = jnp.max(logits, axis=-1, keepdims=True)
    e = jnp.exp(logits - m)
    s = jnp.sum(e, axis=-1, keepdims=True)
    probs_ref_slice[...] = e / s

    iota = lax.broadcasted_iota(jnp.int32, logits.shape, 1)
    i1 = jnp.min(
        jnp.where(logits == m, iota, NUM_EXPERTS), axis=-1, keepdims=True
    )
    masked = jnp.where(iota == i1, -jnp.inf, logits)
    m2 = jnp.max(masked, axis=-1, keepdims=True)
    i2 = jnp.min(
        jnp.where(masked == m2, iota, NUM_EXPERTS), axis=-1, keepdims=True
    )

    e2 = jnp.exp(m2 - m)
    p1 = 1.0 / (1.0 + e2)
    p2 = e2 / (1.0 + e2)

    topk_ref_slice[...] = jnp.concatenate([p1, p2], axis=-1)
    idx_ref_slice[...] = jnp.concatenate([i1, i2], axis=-1)


def _router_kernel(xa_ref, xb_ref, w_ref, b_ref, probs_ref, topk_ref, idx_ref):
    w = w_ref[...]
    bvec = b_ref[...]
    la = jnp.dot(xa_ref[...], w, preferred_element_type=jnp.float32) + bvec
    _top2(la, probs_ref.at[pl.ds(0, BTS), :], topk_ref.at[pl.ds(0, BTS), :],
          idx_ref.at[pl.ds(0, BTS), :])
    lb = jnp.dot(xb_ref[...], w, preferred_element_type=jnp.float32) + bvec
    _top2(lb, probs_ref.at[pl.ds(BTS, BTS), :], topk_ref.at[pl.ds(BTS, BTS), :],
          idx_ref.at[pl.ds(BTS, BTS), :])


@jax.jit
def kernel(inputs, W, b):
    b2 = b.reshape(1, NUM_EXPERTS)
    grid = (NUM_TOKENS // BT,)
    probs, topk, idx = pl.pallas_call(
        _router_kernel,
        grid=grid,
        in_specs=[
            pl.BlockSpec((BTS, D_MODEL), lambda i: (2 * i, 0)),
            pl.BlockSpec((BTS, D_MODEL), lambda i: (2 * i + 1, 0)),
            pl.BlockSpec((D_MODEL, NUM_EXPERTS), lambda i: (0, 0)),
            pl.BlockSpec((1, NUM_EXPERTS), lambda i: (0, 0)),
        ],
        out_specs=[
            pl.BlockSpec((BT, NUM_EXPERTS), lambda i: (i, 0)),
            pl.BlockSpec((BT, TOP_K), lambda i: (i, 0)),
            pl.BlockSpec((BT, TOP_K), lambda i: (i, 0)),
        ],
        out_shape=[
            jax.ShapeDtypeStruct((NUM_TOKENS, NUM_EXPERTS), jnp.float32),
            jax.ShapeDtypeStruct((NUM_TOKENS, TOP_K), jnp.float32),
            jax.ShapeDtypeStruct((NUM_TOKENS, TOP_K), jnp.int32),
        ],
    )(inputs, inputs, W, b2)
    return (topk, idx, probs)
